# baseline (speedup 1.0000x reference)
import jax
import jax.numpy as jnp
from jax.experimental import pallas as pl
from jax.experimental.pallas import tpu as pltpu


def _make_kernel(EJ):
    def _fused_kernel(x_ref, wp_ref, wft_ref, wfb_ref, bp_ref, bf_ref,
                      kn_ref, o_ref, wc_ref, bias_ref):
        j = pl.program_id(1)
        cols = pl.ds(j * EJ, EJ)

        @pl.when(pl.program_id(0) == 0)
        def _prep():
            wft = wft_ref[...]
            wc_ref[:, cols] = jnp.dot(
                wp_ref[...].astype(jnp.bfloat16),
                wft.astype(jnp.bfloat16),
                preferred_element_type=jnp.float32,
            ).astype(jnp.bfloat16)
            mean_k = jnp.mean(kn_ref[...], axis=0, keepdims=True)
            bias_ref[:, cols] = (
                jnp.dot(bp_ref[...], wft, preferred_element_type=jnp.float32)
                + jnp.dot(mean_k, wfb_ref[...],
                          preferred_element_type=jnp.float32)
                + bf_ref[...]
            )

        o_ref[...] = (
            jnp.dot(x_ref[...].astype(jnp.bfloat16), wc_ref[:, cols],
                    preferred_element_type=jnp.float32)
            + bias_ref[:, cols]
        )

    return _fused_kernel


def kernel(inputs, W_proj, b_proj, W_fus, b_fus, knowledge):
    B, S, D = inputs.shape
    E = W_proj.shape[1]
    BS = B * S
    TILE = 1024
    J = 2
    EJ = E // J

    x2 = inputs.reshape(BS, D)
    bp_row = b_proj.reshape(1, E)
    bf_row = b_fus.reshape(1, E)

    def _w_cols(i, j):
        return jnp.where(i < 1, j, 0)

    out = pl.pallas_call(
        _make_kernel(EJ),
        grid=(BS // TILE, J),
        in_specs=[
            pl.BlockSpec((TILE, D), lambda i, j: (i, 0)),
            pl.BlockSpec((D, E), lambda i, j: (0, 0)),
            pl.BlockSpec((E, EJ), lambda i, j: (0, _w_cols(i, j))),
            pl.BlockSpec((E, EJ), lambda i, j: (1, _w_cols(i, j))),
            pl.BlockSpec((1, E), lambda i, j: (0, 0)),
            pl.BlockSpec((1, EJ), lambda i, j: (0, _w_cols(i, j))),
            pl.BlockSpec(knowledge.shape, lambda i, j: (0, 0)),
        ],
        out_specs=pl.BlockSpec((TILE, EJ), lambda i, j: (i, j)),
        out_shape=jax.ShapeDtypeStruct((BS, E), jnp.float32),
        scratch_shapes=[
            pltpu.VMEM((D, E), jnp.bfloat16),
            pltpu.VMEM((1, E), jnp.float32),
        ],
        compiler_params=pltpu.CompilerParams(
            dimension_semantics=("arbitrary", "arbitrary"),
        ),
    )(x2, W_proj, W_fus, W_fus, bp_row, bf_row, knowledge)

    return out.reshape(B, S, E)


# speedup vs baseline: 1.2024x; 1.2024x over previous
import jax
import jax.numpy as jnp
from jax.experimental import pallas as pl
from jax.experimental.pallas import tpu as pltpu


def _fused_kernel(x_ref, wp_ref, wft_ref, wfb_ref, bp_ref, bf_ref, kn_ref,
                  o_ref, wc_ref, bias_ref):
    @pl.when(pl.program_id(0) == 0)
    def _prep():
        wft = wft_ref[...]
        wc_ref[...] = jnp.dot(wp_ref[...].astype(jnp.bfloat16),
                              wft.astype(jnp.bfloat16),
                              preferred_element_type=jnp.float32
                              ).astype(jnp.bfloat16)
        mean_k = jnp.mean(kn_ref[...], axis=0, keepdims=True)
        bias_ref[...] = (
            jnp.dot(bp_ref[...], wft, preferred_element_type=jnp.float32)
            + jnp.dot(mean_k, wfb_ref[...], preferred_element_type=jnp.float32)
            + bf_ref[...]
        )

    o_ref[...] = (
        jnp.dot(x_ref[...].astype(jnp.bfloat16), wc_ref[...],
                preferred_element_type=jnp.float32)
        + bias_ref[...]
    )


def kernel(inputs, W_proj, b_proj, W_fus, b_fus, knowledge):
    B, S, D = inputs.shape
    E = W_proj.shape[1]
    BS = B * S
    TILE = 1024

    x2 = inputs.reshape(BS, D)
    bp_row = b_proj.reshape(1, E)
    bf_row = b_fus.reshape(1, E)

    out = pl.pallas_call(
        _fused_kernel,
        grid=(BS // TILE,),
        in_specs=[
            pl.BlockSpec((TILE, D), lambda i: (i, 0)),
            pl.BlockSpec((D, E), lambda i: (0, 0)),
            pl.BlockSpec((E, E), lambda i: (0, 0)),
            pl.BlockSpec((E, E), lambda i: (1, 0)),
            pl.BlockSpec((1, E), lambda i: (0, 0)),
            pl.BlockSpec((1, E), lambda i: (0, 0)),
            pl.BlockSpec(knowledge.shape, lambda i: (0, 0)),
        ],
        out_specs=pl.BlockSpec((TILE, E), lambda i: (i, 0)),
        out_shape=jax.ShapeDtypeStruct((BS, E), jnp.float32),
        scratch_shapes=[
            pltpu.VMEM((D, E), jnp.bfloat16),
            pltpu.VMEM((1, E), jnp.float32),
        ],
        compiler_params=pltpu.CompilerParams(
            dimension_semantics=("arbitrary",),
        ),
    )(x2, W_proj, W_fus, W_fus, bp_row, bf_row, knowledge)

    return out.reshape(B, S, E)


# speedup vs baseline: 1.2087x; 1.0052x over previous
import jax
import jax.numpy as jnp
from jax.experimental import pallas as pl
from jax.experimental.pallas import tpu as pltpu


def _fused_kernel(x_ref, wp_ref, wft_ref, wfb_ref, bp_ref, bf_ref, kn_ref,
                  o_ref, wc_ref, bias_ref):
    @pl.when(pl.program_id(0) == 0)
    def _prep():
        wft = wft_ref[...]
        wc_ref[...] = jnp.dot(wp_ref[...].astype(jnp.bfloat16),
                              wft.astype(jnp.bfloat16),
                              preferred_element_type=jnp.float32
                              ).astype(jnp.bfloat16)
        mean_k = jnp.mean(kn_ref[...], axis=0, keepdims=True)
        bias_ref[...] = (
            jnp.dot(bp_ref[...], wft, preferred_element_type=jnp.float32)
            + jnp.dot(mean_k, wfb_ref[...], preferred_element_type=jnp.float32)
            + bf_ref[...]
        )

    @pl.when(pl.program_id(0) > 0)
    def _main():
        o_ref[...] = (
            jnp.dot(x_ref[...].astype(jnp.bfloat16), wc_ref[...],
                    preferred_element_type=jnp.float32)
            + bias_ref[...]
        )


def kernel(inputs, W_proj, b_proj, W_fus, b_fus, knowledge):
    B, S, D = inputs.shape
    E = W_proj.shape[1]
    BS = B * S
    TILE = 1024

    x2 = inputs.reshape(BS, D)
    bp_row = b_proj.reshape(1, E)
    bf_row = b_fus.reshape(1, E)

    out = pl.pallas_call(
        _fused_kernel,
        grid=(BS // TILE + 1,),
        in_specs=[
            pl.BlockSpec((TILE, D), lambda i: (jnp.maximum(i - 1, 0), 0)),
            pl.BlockSpec((D, E), lambda i: (0, 0)),
            pl.BlockSpec((E, E), lambda i: (0, 0)),
            pl.BlockSpec((E, E), lambda i: (1, 0)),
            pl.BlockSpec((1, E), lambda i: (0, 0)),
            pl.BlockSpec((1, E), lambda i: (0, 0)),
            pl.BlockSpec(knowledge.shape, lambda i: (0, 0)),
        ],
        out_specs=pl.BlockSpec((TILE, E), lambda i: (jnp.maximum(i - 1, 0), 0)),
        out_shape=jax.ShapeDtypeStruct((BS, E), jnp.float32),
        scratch_shapes=[
            pltpu.VMEM((D, E), jnp.bfloat16),
            pltpu.VMEM((1, E), jnp.float32),
        ],
        compiler_params=pltpu.CompilerParams(
            dimension_semantics=("arbitrary",),
        ),
    )(x2, W_proj, W_fus, W_fus, bp_row, bf_row, knowledge)

    return out.reshape(B, S, E)
